# baseline (speedup 1.0000x reference)
import functools

import jax
import jax.numpy as jnp
from jax import lax
from jax.experimental import pallas as pl
from jax.experimental.pallas import tpu as pltpu
from jax.experimental.pallas import tpu_sc as plsc

VOCAB = 50304
DIM = 768
NEMB = 6
BATCH = 4
SEQ = 2048

NW = 32
ROWS = BATCH * SEQ
RPW = ROWS // NW
CHUNK = 64
NCHUNK = RPW // CHUNK
NBUF = 2

_mesh = plsc.VectorSubcoreMesh(core_axis_name="c", subcore_axis_name="s")


@functools.partial(
    pl.kernel,
    mesh=_mesh,
    out_type=jax.ShapeDtypeStruct((ROWS, DIM), jnp.float32),
    scratch_types=(
        [pltpu.VMEM((RPW,), jnp.int32)]
        + [pltpu.VMEM((CHUNK, DIM), jnp.float32)] * NBUF
        + [pltpu.SemaphoreType.DMA] * (2 * NBUF)
    ),
)
def _gather1(idx_hbm, tab_hbm, out, idx_v, *rest):
    bufs = rest[:NBUF]
    gsems = rest[NBUF:2 * NBUF]
    wsems = rest[2 * NBUF:]
    wid = lax.axis_index("s") * 2 + lax.axis_index("c")
    base = wid * RPW
    pltpu.sync_copy(idx_hbm.at[wid], idx_v)

    def gather(step):
        b = step % NBUF
        return pltpu.async_copy(
            tab_hbm.at[idx_v.at[pl.ds(step * CHUNK, CHUNK)]],
            bufs[b], gsems[b])

    def write(step):
        b = step % NBUF
        return pltpu.async_copy(
            bufs[b], out.at[pl.ds(base + step * CHUNK, CHUNK)], wsems[b])

    writes = [None] * NCHUNK
    gathers = [None] * NCHUNK
    for s in range(min(NBUF - 1, NCHUNK)):
        gathers[s] = gather(s)
    for s in range(NCHUNK):
        gathers[s].wait()
        writes[s] = write(s)
        nxt = s + NBUF - 1
        if nxt < NCHUNK:
            if s >= 1:
                writes[s - 1].wait()
            gathers[nxt] = gather(nxt)
    for s in range(max(0, NCHUNK - NBUF), NCHUNK):
        writes[s].wait()


def kernel(inputs, tables):
    flat = inputs.reshape(-1).astype(jnp.int32)
    offs = (jnp.arange(NEMB, dtype=jnp.int32) * VOCAB)[:, None]
    idx_all = (flat[None, :] + offs).reshape(NEMB, NW, RPW)
    tab = tables.reshape(NEMB * VOCAB, DIM)
    one = (inputs[0, 0] * 0 + 1).astype(jnp.float32)
    ve, dup = [], []
    for t in range(NEMB):
        idx_t = idx_all[t]
        if t >= 2:
            tok = (dup[t - 2].ravel()[0] * 0).astype(jnp.int32)
            idx_t = idx_t + tok
        ve.append(_gather1(idx_t, tab).reshape(BATCH, SEQ, DIM))
        dup.append(ve[t] * one)
    return tuple(ve + dup[::-1])


# speedup vs baseline: 1.1236x; 1.1236x over previous
import functools

import jax
import jax.numpy as jnp
from jax import lax
from jax.experimental import pallas as pl
from jax.experimental.pallas import tpu as pltpu
from jax.experimental.pallas import tpu_sc as plsc

VOCAB = 50304
DIM = 768
NEMB = 6
BATCH = 4
SEQ = 2048

NW = 32
ROWS = BATCH * SEQ
RPW = ROWS // NW
CHUNK = 64
NCHUNK = RPW // CHUNK
NBUF = 2

_mesh = plsc.VectorSubcoreMesh(core_axis_name="c", subcore_axis_name="s")


@functools.partial(
    pl.kernel,
    mesh=_mesh,
    out_type=jax.ShapeDtypeStruct((ROWS, DIM), jnp.float32),
    scratch_types=(
        [pltpu.VMEM((RPW,), jnp.int32)]
        + [pltpu.VMEM((CHUNK, DIM), jnp.float32)] * NBUF
        + [pltpu.SemaphoreType.DMA] * (2 * NBUF)
    ),
)
def _gather1(idx_hbm, tab_hbm, out, idx_v, *rest):
    bufs = rest[:NBUF]
    gsems = rest[NBUF:2 * NBUF]
    wsems = rest[2 * NBUF:]
    wid = lax.axis_index("s") * 2 + lax.axis_index("c")
    base = wid * RPW
    pltpu.sync_copy(idx_hbm.at[wid], idx_v)

    def gather(step):
        b = step % NBUF
        return pltpu.async_copy(
            tab_hbm.at[idx_v.at[pl.ds(step * CHUNK, CHUNK)]],
            bufs[b], gsems[b])

    def write(step):
        b = step % NBUF
        return pltpu.async_copy(
            bufs[b], out.at[pl.ds(base + step * CHUNK, CHUNK)], wsems[b])

    writes = [None] * NCHUNK
    gathers = [None] * NCHUNK
    for s in range(min(NBUF - 1, NCHUNK)):
        gathers[s] = gather(s)
    for s in range(NCHUNK):
        gathers[s].wait()
        writes[s] = write(s)
        nxt = s + NBUF - 1
        if nxt < NCHUNK:
            if s >= 1:
                writes[s - 1].wait()
            gathers[nxt] = gather(nxt)
    for s in range(max(0, NCHUNK - NBUF), NCHUNK):
        writes[s].wait()


def kernel(inputs, tables):
    flat = inputs.reshape(-1).astype(jnp.int32)
    offs = (jnp.arange(NEMB, dtype=jnp.int32) * VOCAB)[:, None]
    idx_all = (flat[None, :] + offs).reshape(NEMB, NW, RPW)
    tab = tables.reshape(NEMB * VOCAB, DIM)
    tiny = jnp.where(inputs[0, 0] < 0, jnp.float32(1), jnp.float32(0))
    ve, dup = [], []
    for t in range(NEMB):
        idx_t = idx_all[t]
        if t >= 2:
            idx_t, _ = lax.optimization_barrier((idx_t, dup[t - 2]))
        ve.append(_gather1(idx_t, tab).reshape(BATCH, SEQ, DIM))
        dup.append(ve[t] + tiny)
    return tuple(ve + dup[::-1])


# speedup vs baseline: 1.1711x; 1.0423x over previous
import functools

import jax
import jax.numpy as jnp
from jax import lax
from jax.experimental import pallas as pl
from jax.experimental.pallas import tpu as pltpu
from jax.experimental.pallas import tpu_sc as plsc

VOCAB = 50304
DIM = 768
NEMB = 6
BATCH = 4
SEQ = 2048

NW = 32
ROWS = BATCH * SEQ
RPW = ROWS // NW
CHUNK = 64
NCHUNK = RPW // CHUNK
NBUF = 2

_mesh = plsc.VectorSubcoreMesh(core_axis_name="c", subcore_axis_name="s")


def _make_gather(n_out):
    @functools.partial(
        pl.kernel,
        mesh=_mesh,
        out_type=[jax.ShapeDtypeStruct((ROWS, DIM), jnp.float32)
                  for _ in range(n_out)],
        scratch_types=(
            [pltpu.VMEM((RPW,), jnp.int32)]
            + [pltpu.VMEM((CHUNK, DIM), jnp.float32)] * NBUF
            + [pltpu.SemaphoreType.DMA] * (2 * NBUF)
        ),
    )
    def _gather(idx_hbm, tab_hbm, *rest):
        outs = rest[:n_out]
        idx_v = rest[n_out]
        bufs = rest[n_out + 1:n_out + 1 + NBUF]
        gsems = rest[n_out + 1 + NBUF:n_out + 1 + 2 * NBUF]
        wsems = rest[n_out + 1 + 2 * NBUF:]
        wid = lax.axis_index("s") * 2 + lax.axis_index("c")
        base = wid * RPW
        pltpu.sync_copy(idx_hbm.at[wid], idx_v)

        def gather(step):
            b = step % NBUF
            return pltpu.async_copy(
                tab_hbm.at[idx_v.at[pl.ds(step * CHUNK, CHUNK)]],
                bufs[b], gsems[b])

        def write(step):
            b = step % NBUF
            dst = pl.ds(base + step * CHUNK, CHUNK)
            return tuple(pltpu.async_copy(bufs[b], o.at[dst], wsems[b])
                         for o in outs)

        writes = [None] * NCHUNK
        gathers = [None] * NCHUNK
        for s in range(min(NBUF - 1, NCHUNK)):
            gathers[s] = gather(s)
        for s in range(NCHUNK):
            gathers[s].wait()
            writes[s] = write(s)
            nxt = s + NBUF - 1
            if nxt < NCHUNK:
                if s >= 1:
                    for w in writes[s - 1]:
                        w.wait()
                gathers[nxt] = gather(nxt)
        for s in range(max(0, NCHUNK - NBUF), NCHUNK):
            for w in writes[s]:
                w.wait()

    return _gather


_gather1 = _make_gather(1)
_gather2 = _make_gather(2)


def kernel(inputs, tables):
    flat = inputs.reshape(-1).astype(jnp.int32)
    offs = (jnp.arange(NEMB, dtype=jnp.int32) * VOCAB)[:, None]
    idx_all = (flat[None, :] + offs).reshape(NEMB, NW, RPW)
    tab = tables.reshape(NEMB * VOCAB, DIM)
    tiny = jnp.where(inputs[0, 0] < 0, jnp.float32(1), jnp.float32(0))
    ve, dup = [], []
    for t in range(NEMB - 1):
        idx_t = idx_all[t]
        if t >= 3:
            idx_t, _ = lax.optimization_barrier((idx_t, dup[t - 3]))
        (o,) = _gather1(idx_t, tab)
        ve.append(o.reshape(BATCH, SEQ, DIM))
        dup.append(ve[t] + tiny)
    o5, o5dup = _gather2(idx_all[NEMB - 1], tab)
    ve.append(o5.reshape(BATCH, SEQ, DIM))
    dup.append(o5dup.reshape(BATCH, SEQ, DIM))
    return tuple(ve + dup[::-1])


# speedup vs baseline: 1.3959x; 1.1919x over previous
import functools

import jax
import jax.numpy as jnp
from jax import lax
from jax.experimental import pallas as pl
from jax.experimental.pallas import tpu as pltpu
from jax.experimental.pallas import tpu_sc as plsc

VOCAB = 50304
DIM = 768
NEMB = 6
BATCH = 4
SEQ = 2048

NW = 32
ROWS = BATCH * SEQ
RPW = ROWS // NW
CHUNK = 32
NCHUNK = RPW // CHUNK
NSTEPS = NEMB * NCHUNK
NBUF = 4

_mesh = plsc.VectorSubcoreMesh(core_axis_name="c", subcore_axis_name="s")


@functools.partial(
    pl.kernel,
    mesh=_mesh,
    out_type=[jax.ShapeDtypeStruct((ROWS, DIM), jnp.float32)
              for _ in range(2 * NEMB)],
    scratch_types=(
        [pltpu.VMEM((NEMB, RPW), jnp.int32)]
        + [pltpu.VMEM((CHUNK, DIM), jnp.float32)] * NBUF
        + [pltpu.SemaphoreType.DMA] * (2 * NBUF)
    ),
)
def _gather6(idx_hbm, tab_hbm,
             o0, o1, o2, o3, o4, o5, o6, o7, o8, o9, o10, o11,
             idx_v, *rest):
    outs = (o0, o1, o2, o3, o4, o5, o6, o7, o8, o9, o10, o11)
    bufs = rest[:NBUF]
    gsems = rest[NBUF:2 * NBUF]
    wsems = rest[2 * NBUF:]
    wid = lax.axis_index("s") * 2 + lax.axis_index("c")
    base = wid * RPW
    pltpu.sync_copy(idx_hbm.at[wid], idx_v)

    def gather(step):
        t, ch = divmod(step, NCHUNK)
        b = step % NBUF
        return pltpu.async_copy(
            tab_hbm.at[idx_v.at[t, pl.ds(ch * CHUNK, CHUNK)]],
            bufs[b], gsems[b])

    def write(step):
        t, ch = divmod(step, NCHUNK)
        b = step % NBUF
        dst = pl.ds(base + ch * CHUNK, CHUNK)
        w1_ = pltpu.async_copy(bufs[b], outs[t].at[dst], wsems[b])
        w2_ = pltpu.async_copy(bufs[b], outs[11 - t].at[dst], wsems[b])
        return (w1_, w2_)

    writes = [None] * NSTEPS
    gathers = [None] * NSTEPS
    for s in range(min(NBUF - 1, NSTEPS)):
        gathers[s] = gather(s)
    for s in range(NSTEPS):
        gathers[s].wait()
        writes[s] = write(s)
        nxt = s + NBUF - 1
        if nxt < NSTEPS:
            if s >= 1:
                for w in writes[s - 1]:
                    w.wait()
            gathers[nxt] = gather(nxt)
    for s in range(max(0, NSTEPS - NBUF), NSTEPS):
        for w in writes[s]:
            w.wait()


def kernel(inputs, tables):
    flat = inputs.reshape(-1).astype(jnp.int32)
    offs = (jnp.arange(NEMB, dtype=jnp.int32) * VOCAB)[:, None]
    idx_all = (flat[None, :] + offs).reshape(NEMB, NW, RPW).transpose(1, 0, 2)
    tab = tables.reshape(NEMB * VOCAB, DIM)
    outs = _gather6(idx_all, tab)
    return tuple(o.reshape(BATCH, SEQ, DIM) for o in outs)


# speedup vs baseline: 1.3963x; 1.0003x over previous
import functools

import jax
import jax.numpy as jnp
from jax import lax
from jax.experimental import pallas as pl
from jax.experimental.pallas import tpu as pltpu
from jax.experimental.pallas import tpu_sc as plsc

VOCAB = 50304
DIM = 768
NEMB = 6
BATCH = 4
SEQ = 2048

NW = 32
ROWS = BATCH * SEQ
RPW = ROWS // NW
CHUNK = 64
NCHUNK = RPW // CHUNK
NSTEPS = NEMB * NCHUNK
NBUF = 2

_mesh = plsc.VectorSubcoreMesh(core_axis_name="c", subcore_axis_name="s")


@functools.partial(
    pl.kernel,
    mesh=_mesh,
    out_type=[jax.ShapeDtypeStruct((ROWS, DIM), jnp.float32)
              for _ in range(2 * NEMB)],
    scratch_types=(
        [pltpu.VMEM((NEMB, RPW), jnp.int32)]
        + [pltpu.VMEM((CHUNK, DIM), jnp.float32)] * NBUF
        + [pltpu.SemaphoreType.DMA] * (2 * NBUF)
    ),
)
def _gather6(idx_hbm, tab_hbm,
             o0, o1, o2, o3, o4, o5, o6, o7, o8, o9, o10, o11,
             idx_v, *rest):
    outs = (o0, o1, o2, o3, o4, o5, o6, o7, o8, o9, o10, o11)
    bufs = rest[:NBUF]
    gsems = rest[NBUF:2 * NBUF]
    wsems = rest[2 * NBUF:]
    wid = lax.axis_index("s") * 2 + lax.axis_index("c")
    base = wid * RPW
    pltpu.sync_copy(idx_hbm.at[wid], idx_v)

    def gather(step):
        t, ch = divmod(step, NCHUNK)
        b = step % NBUF
        return pltpu.async_copy(
            tab_hbm.at[idx_v.at[t, pl.ds(ch * CHUNK, CHUNK)]],
            bufs[b], gsems[b])

    def write(step):
        t, ch = divmod(step, NCHUNK)
        b = step % NBUF
        dst = pl.ds(base + ch * CHUNK, CHUNK)
        w1_ = pltpu.async_copy(bufs[b], outs[t].at[dst], wsems[b])
        w2_ = pltpu.async_copy(bufs[b], outs[11 - t].at[dst], wsems[b])
        return (w1_, w2_)

    writes = [None] * NSTEPS
    gathers = [None] * NSTEPS
    for s in range(min(NBUF - 1, NSTEPS)):
        gathers[s] = gather(s)
    for s in range(NSTEPS):
        gathers[s].wait()
        writes[s] = write(s)
        nxt = s + NBUF - 1
        if nxt < NSTEPS:
            if s >= 1:
                for w in writes[s - 1]:
                    w.wait()
            gathers[nxt] = gather(nxt)
    for s in range(max(0, NSTEPS - NBUF), NSTEPS):
        for w in writes[s]:
            w.wait()


def kernel(inputs, tables):
    flat = inputs.reshape(-1).astype(jnp.int32)
    offs = (jnp.arange(NEMB, dtype=jnp.int32) * VOCAB)[:, None]
    idx_all = (flat[None, :] + offs).reshape(NEMB, NW, RPW).transpose(1, 0, 2)
    tab = tables.reshape(NEMB * VOCAB, DIM)
    outs = _gather6(idx_all, tab)
    return tuple(o.reshape(BATCH, SEQ, DIM) for o in outs)
